# baseline (speedup 1.0000x reference)
import functools

import jax
import jax.numpy as jnp
from jax import lax
from jax.experimental import pallas as pl
from jax.experimental.pallas import tpu as pltpu
from jax.experimental.pallas import tpu_sc as plsc

N = 10000
E = 320000
D = 128
NC = 2
NS = 16
NW = NC * NS
EPW = E // NW
C = 80
NCH = -(-EPW // C)
PADW = (NCH + 1) * C - EPW
RB = 10
TCB = N // RB
ZR = 624
REM_OFF = NS * ZR
REM = N - REM_OFF


def _zst_body(h_ref, wfc_ref, wa_ref, z_ref, st_ref):
    h = h_ref[...]
    z = lax.dot_general(h, wfc_ref[...], (((1,), (1,)), ((), ())),
                        preferred_element_type=jnp.float32)
    z_ref[...] = z
    st_ref[...] = lax.dot_general(z, wa_ref[...], (((1,), (1,)), ((), ())),
                                  preferred_element_type=jnp.float32)


def _zst(h, wfc, wa2):
    return pl.pallas_call(
        _zst_body,
        grid=(RB,),
        in_specs=[
            pl.BlockSpec((TCB, D), lambda i: (i, 0)),
            pl.BlockSpec((D, D), lambda i: (0, 0)),
            pl.BlockSpec((2, D), lambda i: (0, 0)),
        ],
        out_specs=[
            pl.BlockSpec((TCB, D), lambda i: (i, 0)),
            pl.BlockSpec((TCB, 2), lambda i: (i, 0)),
        ],
        out_shape=[
            jax.ShapeDtypeStruct((N, D), jnp.float32),
            jax.ShapeDtypeStruct((N, 2), jnp.float32),
        ],
    )(h, wfc, wa2)


_SC_MESH = plsc.VectorSubcoreMesh(core_axis_name="c", subcore_axis_name="s")


@functools.partial(
    pl.kernel,
    mesh=_SC_MESH,
    out_type=(
        jax.ShapeDtypeStruct((NC, N, D), jnp.float32),
        jax.ShapeDtypeStruct((N,), jnp.float32),
        jax.ShapeDtypeStruct((N,), jnp.float32),
    ),
    scratch_types=[
        pltpu.VMEM(((NCH + 1) * C,), jnp.int32),
        pltpu.VMEM((NCH + 1, C), jnp.int32),
        pltpu.VMEM((C,), jnp.float32),
        pltpu.VMEM((C,), jnp.float32),
        pltpu.VMEM((C,), jnp.float32),
        pltpu.VMEM((C,), jnp.float32),
        pltpu.VMEM((C,), jnp.float32),
        pltpu.VMEM((C,), jnp.float32),
        pltpu.VMEM((C, D), jnp.float32),
        pltpu.VMEM((C, D), jnp.float32),
        pltpu.VMEM_SHARED((N, D), jnp.float32),
        pltpu.VMEM_SHARED((N,), jnp.float32),
        pltpu.SemaphoreType.DMA,
        pltpu.SemaphoreType.DMA,
        pltpu.SemaphoreType.DMA,
        pltpu.SemaphoreType.DMA,
        pltpu.SemaphoreType.DMA,
        pltpu.SemaphoreType.DMA,
        pltpu.SemaphoreType.DMA,
        pltpu.SemaphoreType.DMA,
    ],
    compiler_params=pltpu.CompilerParams(needs_layout_passes=False),
)
def _gat_sc(src_hbm, dst2_hbm, s_hbm, t_hbm, z_hbm, zrow_hbm, zden_hbm,
            pout_hbm, pden0_hbm, pden1_hbm,
            src_f, dst2_v, sA, sB, tA, tB, wA, wB, rowsA, rowsB,
            out_sh, den_sh,
            semzA, semzB, semstA, semstB, semrA, semrB, semdA, semdB):
    cid = lax.axis_index("c")
    sid = lax.axis_index("s")
    wid = cid * NS + sid

    pltpu.sync_copy(src_hbm.at[wid], src_f)
    pltpu.sync_copy(dst2_hbm.at[wid], dst2_v)

    pltpu.sync_copy(zrow_hbm.at[pl.ds(sid * ZR, ZR)],
                    out_sh.at[pl.ds(sid * ZR, ZR)])

    @pl.when(sid == NS - 1)
    def _():
        pltpu.sync_copy(zrow_hbm.at[pl.ds(REM_OFF, REM)],
                        out_sh.at[pl.ds(REM_OFF, REM)])

    @pl.when(sid == 0)
    def _():
        pltpu.sync_copy(zden_hbm, den_sh)

    plsc.subcore_barrier()

    zeros16 = jnp.zeros((16,), jnp.int32)

    bufs = (
        (sA, tA, wA, rowsA, semzA, semstA, semrA, semdA),
        (sB, tB, wB, rowsB, semzB, semstB, semrB, semdB),
    )

    H = C // 2

    def issue_gathers(c, bx):
        sb, tb, _, rows, semz, semst, _, _ = bx
        sidx = src_f.at[pl.ds(c * C, C)]
        pltpu.async_copy(z_hbm.at[src_f.at[pl.ds(c * C, H)]],
                         rows.at[pl.ds(0, H)], semz)
        pltpu.async_copy(z_hbm.at[src_f.at[pl.ds(c * C + H, H)]],
                         rows.at[pl.ds(H, H)], semz)
        pltpu.async_copy(s_hbm.at[sidx], sb, semst)
        pltpu.async_copy(t_hbm.at[dst2_v.at[c]], tb, semst)

    def wait_st(bx):
        sb, tb, _, _, _, semst, _, _ = bx
        pltpu.make_async_copy(s_hbm.at[pl.ds(0, C)], sb, semst).wait()
        pltpu.make_async_copy(t_hbm.at[pl.ds(0, C)], tb, semst).wait()

    def wait_z(bx):
        rows, semz = bx[3], bx[4]
        pltpu.make_async_copy(z_hbm.at[pl.ds(0, C)], rows, semz).wait()

    iota16 = jnp.arange(16, dtype=jnp.int32)

    def compute_w(c, bx):
        sb, tb, wb = bx[0], bx[1], bx[2]
        ebase = c * C
        for g in range(C // 16):
            sl = pl.ds(g * 16, 16)
            e = sb[sl] + tb[sl]
            e = jnp.where(e > 0.0, e, e * 0.01)
            w = jnp.exp(e)
            eid = iota16 + (ebase + g * 16)
            wb[sl] = jnp.where(eid < EPW, w, 0.0)

    UNROLL = 8

    def scale_rows(bx):
        wb, rows = bx[2], bx[3]

        def scale(j, carry):
            j0 = j * UNROLL
            for u in range(UNROLL):
                r = j0 + u
                wsplat = plsc.load_gather(wb, [zeros16 + r])
                for kk in range(D // 16):
                    sl = pl.ds(kk * 16, 16)
                    rows[r, sl] = rows[r, sl] * wsplat
            return carry

        lax.fori_loop(0, C // UNROLL, scale, 0)

    def issue_scatters(c, bx):
        wb, rows, semr, semd = bx[2], bx[3], bx[6], bx[7]
        didx = dst2_v.at[c]
        pltpu.async_copy(rows, out_sh.at[didx], semr, add=True)
        pltpu.async_copy(wb, den_sh.at[didx], semd, add=True)

    def wait_scatters(bx):
        wb, rows, semr, semd = bx[2], bx[3], bx[6], bx[7]
        pltpu.make_async_copy(rows, z_hbm.at[pl.ds(0, C)], semr).wait()
        pltpu.make_async_copy(wb, s_hbm.at[pl.ds(0, C)], semd).wait()

    def step(c, bx, by):
        wait_st(bx)
        compute_w(c, bx)
        wait_z(bx)
        wait_scatters(by)
        issue_gathers(c + 1, by)
        scale_rows(bx)
        issue_scatters(c, bx)

    issue_gathers(0, bufs[0])
    wait_st(bufs[0])
    compute_w(0, bufs[0])
    wait_z(bufs[0])
    issue_gathers(1, bufs[1])
    scale_rows(bufs[0])
    issue_scatters(0, bufs[0])

    def pair(p, carry):
        step(2 * p + 1, bufs[1], bufs[0])
        step(2 * p + 2, bufs[0], bufs[1])
        return carry

    lax.fori_loop(0, (NCH - 1) // 2, pair, 0)

    wait_scatters(bufs[0])
    wait_st(bufs[1])
    wait_z(bufs[1])

    plsc.subcore_barrier()
    pltpu.sync_copy(out_sh.at[pl.ds(sid * ZR, ZR)],
                    pout_hbm.at[cid, pl.ds(sid * ZR, ZR)])

    @pl.when(sid == NS - 1)
    def _():
        pltpu.sync_copy(out_sh.at[pl.ds(REM_OFF, REM)],
                        pout_hbm.at[cid, pl.ds(REM_OFF, REM)])

    @pl.when(jnp.logical_and(sid == 0, cid == 0))
    def _():
        pltpu.sync_copy(den_sh, pden0_hbm)

    @pl.when(jnp.logical_and(sid == 0, cid == 1))
    def _():
        pltpu.sync_copy(den_sh, pden1_hbm)


def _fin_body(pout_ref, pd0_ref, pd1_ref, out_ref):
    p = pout_ref[0] + pout_ref[1]
    d = pd0_ref[...] + pd1_ref[...]
    out_ref[...] = p / jnp.maximum(d, 1e-16)


def _finish(pout, pd0, pd1):
    return pl.pallas_call(
        _fin_body,
        grid=(RB,),
        in_specs=[
            pl.BlockSpec((NC, TCB, D), lambda i: (0, i, 0)),
            pl.BlockSpec((TCB, 1), lambda i: (i, 0)),
            pl.BlockSpec((TCB, 1), lambda i: (i, 0)),
        ],
        out_specs=pl.BlockSpec((TCB, D), lambda i: (i, 0)),
        out_shape=jax.ShapeDtypeStruct((N, D), jnp.float32),
    )(pout, pd0, pd1)


def kernel(h, edge_index, W_fc, W_attn):
    z, st = _zst(h, W_fc, W_attn.reshape(2, D))
    pad = jnp.zeros((NW, PADW), jnp.int32)
    src = jnp.concatenate([edge_index[0].reshape(NW, EPW), pad], axis=1)
    dst2 = jnp.concatenate([edge_index[1].reshape(NW, EPW), pad],
                           axis=1).reshape(NW, NCH + 1, C)
    zrow = jnp.zeros((N, D), jnp.float32)
    zden = jnp.zeros((N,), jnp.float32)
    pout, pden0, pden1 = _gat_sc(src, dst2, st[:, 0], st[:, 1], z,
                                 zrow, zden)
    return _finish(pout, pden0.reshape(N, 1), pden1.reshape(N, 1))


# speedup vs baseline: 1.0003x; 1.0003x over previous
import functools

import jax
import jax.numpy as jnp
from jax import lax
from jax.experimental import pallas as pl
from jax.experimental.pallas import tpu as pltpu
from jax.experimental.pallas import tpu_sc as plsc

N = 10000
E = 320000
D = 128
NC = 2
NS = 16
NW = NC * NS
EPW = E // NW
C = 80
NCH = -(-EPW // C)
PADW = (NCH + 1) * C - EPW
RB = 10
TCB = N // RB
ZR = 624
REM_OFF = NS * ZR
REM = N - REM_OFF


def _zst_body(h_ref, wfc_ref, wa_ref, z_ref, st_ref):
    h = h_ref[...]
    z = lax.dot_general(h, wfc_ref[...], (((1,), (1,)), ((), ())),
                        preferred_element_type=jnp.float32)
    z_ref[...] = z
    st_ref[...] = lax.dot_general(z, wa_ref[...], (((1,), (1,)), ((), ())),
                                  preferred_element_type=jnp.float32)


def _zst(h, wfc, wa2):
    return pl.pallas_call(
        _zst_body,
        grid=(RB,),
        in_specs=[
            pl.BlockSpec((TCB, D), lambda i: (i, 0)),
            pl.BlockSpec((D, D), lambda i: (0, 0)),
            pl.BlockSpec((2, D), lambda i: (0, 0)),
        ],
        out_specs=[
            pl.BlockSpec((TCB, D), lambda i: (i, 0)),
            pl.BlockSpec((TCB, 2), lambda i: (i, 0)),
        ],
        out_shape=[
            jax.ShapeDtypeStruct((N, D), jnp.float32),
            jax.ShapeDtypeStruct((N, 2), jnp.float32),
        ],
    )(h, wfc, wa2)


_SC_MESH = plsc.VectorSubcoreMesh(core_axis_name="c", subcore_axis_name="s")


@functools.partial(
    pl.kernel,
    mesh=_SC_MESH,
    out_type=(
        jax.ShapeDtypeStruct((NC, N, D), jnp.float32),
        jax.ShapeDtypeStruct((N,), jnp.float32),
        jax.ShapeDtypeStruct((N,), jnp.float32),
    ),
    scratch_types=[
        pltpu.VMEM(((NCH + 1) * C,), jnp.int32),
        pltpu.VMEM((NCH + 1, C), jnp.int32),
        pltpu.VMEM((C,), jnp.float32),
        pltpu.VMEM((C,), jnp.float32),
        pltpu.VMEM((C,), jnp.float32),
        pltpu.VMEM((C,), jnp.float32),
        pltpu.VMEM((C,), jnp.float32),
        pltpu.VMEM((C,), jnp.float32),
        pltpu.VMEM((C, D), jnp.float32),
        pltpu.VMEM((C, D), jnp.float32),
        pltpu.VMEM_SHARED((N, D), jnp.float32),
        pltpu.VMEM_SHARED((N,), jnp.float32),
        pltpu.SemaphoreType.DMA,
        pltpu.SemaphoreType.DMA,
        pltpu.SemaphoreType.DMA,
        pltpu.SemaphoreType.DMA,
        pltpu.SemaphoreType.DMA,
        pltpu.SemaphoreType.DMA,
        pltpu.SemaphoreType.DMA,
        pltpu.SemaphoreType.DMA,
    ],
    compiler_params=pltpu.CompilerParams(needs_layout_passes=False),
)
def _gat_sc(src_hbm, dst2_hbm, s_hbm, t_hbm, z_hbm, zrow_hbm, zden_hbm,
            pout_hbm, pden0_hbm, pden1_hbm,
            src_f, dst2_v, sA, sB, tA, tB, wA, wB, rowsA, rowsB,
            out_sh, den_sh,
            semzA, semzB, semstA, semstB, semrA, semrB, semdA, semdB):
    cid = lax.axis_index("c")
    sid = lax.axis_index("s")
    wid = cid * NS + sid

    pltpu.sync_copy(src_hbm.at[wid], src_f)
    pltpu.sync_copy(dst2_hbm.at[wid], dst2_v)

    pltpu.sync_copy(zrow_hbm.at[pl.ds(sid * ZR, ZR)],
                    out_sh.at[pl.ds(sid * ZR, ZR)])

    @pl.when(sid == NS - 1)
    def _():
        pltpu.sync_copy(zrow_hbm.at[pl.ds(REM_OFF, REM)],
                        out_sh.at[pl.ds(REM_OFF, REM)])

    @pl.when(sid == 0)
    def _():
        pltpu.sync_copy(zden_hbm, den_sh)

    plsc.subcore_barrier()

    zeros16 = jnp.zeros((16,), jnp.int32)

    bufs = (
        (sA, tA, wA, rowsA, semzA, semstA, semrA, semdA),
        (sB, tB, wB, rowsB, semzB, semstB, semrB, semdB),
    )

    def issue_gathers(c, bx):
        sb, tb, _, rows, semz, semst, _, _ = bx
        sidx = src_f.at[pl.ds(c * C, C)]
        pltpu.async_copy(z_hbm.at[sidx], rows, semz)
        pltpu.async_copy(s_hbm.at[sidx], sb, semst)
        pltpu.async_copy(t_hbm.at[dst2_v.at[c]], tb, semst)

    def wait_st(bx):
        sb, tb, _, _, _, semst, _, _ = bx
        pltpu.make_async_copy(s_hbm.at[pl.ds(0, C)], sb, semst).wait()
        pltpu.make_async_copy(t_hbm.at[pl.ds(0, C)], tb, semst).wait()

    def wait_z(bx):
        rows, semz = bx[3], bx[4]
        pltpu.make_async_copy(z_hbm.at[pl.ds(0, C)], rows, semz).wait()

    iota16 = jnp.arange(16, dtype=jnp.int32)

    def compute_w(c, bx):
        sb, tb, wb = bx[0], bx[1], bx[2]
        ebase = c * C
        for g in range(C // 16):
            sl = pl.ds(g * 16, 16)
            e = sb[sl] + tb[sl]
            e = jnp.where(e > 0.0, e, e * 0.01)
            w = jnp.exp(e)
            eid = iota16 + (ebase + g * 16)
            wb[sl] = jnp.where(eid < EPW, w, 0.0)

    UNROLL = 8

    def scale_rows(bx):
        wb, rows = bx[2], bx[3]

        def scale(j, carry):
            j0 = j * UNROLL
            for u in range(UNROLL):
                r = j0 + u
                wsplat = plsc.load_gather(wb, [zeros16 + r])
                for kk in range(D // 16):
                    sl = pl.ds(kk * 16, 16)
                    rows[r, sl] = rows[r, sl] * wsplat
            return carry

        lax.fori_loop(0, C // UNROLL, scale, 0)

    def issue_scatters(c, bx):
        wb, rows, semr, semd = bx[2], bx[3], bx[6], bx[7]
        didx = dst2_v.at[c]
        pltpu.async_copy(rows, out_sh.at[didx], semr, add=True)
        pltpu.async_copy(wb, den_sh.at[didx], semd, add=True)

    def wait_scatters(bx):
        wb, rows, semr, semd = bx[2], bx[3], bx[6], bx[7]
        pltpu.make_async_copy(rows, z_hbm.at[pl.ds(0, C)], semr).wait()
        pltpu.make_async_copy(wb, s_hbm.at[pl.ds(0, C)], semd).wait()

    def step(c, bx, by):
        wait_st(bx)
        compute_w(c, bx)
        wait_z(bx)
        wait_scatters(by)
        issue_gathers(c + 1, by)
        scale_rows(bx)
        issue_scatters(c, bx)

    issue_gathers(0, bufs[0])
    wait_st(bufs[0])
    compute_w(0, bufs[0])
    wait_z(bufs[0])
    issue_gathers(1, bufs[1])
    scale_rows(bufs[0])
    issue_scatters(0, bufs[0])

    def pair(p, carry):
        step(2 * p + 1, bufs[1], bufs[0])
        step(2 * p + 2, bufs[0], bufs[1])
        return carry

    lax.fori_loop(0, (NCH - 1) // 2, pair, 0)

    wait_scatters(bufs[0])
    wait_st(bufs[1])
    wait_z(bufs[1])

    plsc.subcore_barrier()
    pltpu.sync_copy(out_sh.at[pl.ds(sid * ZR, ZR)],
                    pout_hbm.at[cid, pl.ds(sid * ZR, ZR)])

    @pl.when(sid == NS - 1)
    def _():
        pltpu.sync_copy(out_sh.at[pl.ds(REM_OFF, REM)],
                        pout_hbm.at[cid, pl.ds(REM_OFF, REM)])

    @pl.when(jnp.logical_and(sid == 0, cid == 0))
    def _():
        pltpu.sync_copy(den_sh, pden0_hbm)

    @pl.when(jnp.logical_and(sid == 0, cid == 1))
    def _():
        pltpu.sync_copy(den_sh, pden1_hbm)


def _fin_body(pout_ref, pd0_ref, pd1_ref, out_ref):
    p = pout_ref[0] + pout_ref[1]
    d = pd0_ref[...] + pd1_ref[...]
    out_ref[...] = p / jnp.maximum(d, 1e-16)


def _finish(pout, pd0, pd1):
    return pl.pallas_call(
        _fin_body,
        grid=(RB,),
        in_specs=[
            pl.BlockSpec((NC, TCB, D), lambda i: (0, i, 0)),
            pl.BlockSpec((TCB, 1), lambda i: (i, 0)),
            pl.BlockSpec((TCB, 1), lambda i: (i, 0)),
        ],
        out_specs=pl.BlockSpec((TCB, D), lambda i: (i, 0)),
        out_shape=jax.ShapeDtypeStruct((N, D), jnp.float32),
    )(pout, pd0, pd1)


def kernel(h, edge_index, W_fc, W_attn):
    z, st = _zst(h, W_fc, W_attn.reshape(2, D))
    pad = jnp.zeros((NW, PADW), jnp.int32)
    src = jnp.concatenate([edge_index[0].reshape(NW, EPW), pad], axis=1)
    dst2 = jnp.concatenate([edge_index[1].reshape(NW, EPW), pad],
                           axis=1).reshape(NW, NCH + 1, C)
    zrow = jnp.zeros((N, D), jnp.float32)
    zden = jnp.zeros((N,), jnp.float32)
    pout, pden0, pden1 = _gat_sc(src, dst2, st[:, 0], st[:, 1], z,
                                 zrow, zden)
    return _finish(pout, pden0.reshape(N, 1), pden1.reshape(N, 1))


# speedup vs baseline: 1.0033x; 1.0029x over previous
import functools

import jax
import jax.numpy as jnp
from jax import lax
from jax.experimental import pallas as pl
from jax.experimental.pallas import tpu as pltpu
from jax.experimental.pallas import tpu_sc as plsc

N = 10000
E = 320000
D = 128
NC = 2
NS = 16
NW = NC * NS
EPW = E // NW
C = 80
NCH = -(-EPW // C)
PADW = (NCH + 1) * C - EPW
RB = 10
TCB = N // RB
ZR = 624
REM_OFF = NS * ZR
REM = N - REM_OFF


def _zst_body(h_ref, wfc_ref, wa_ref, z_ref, st_ref):
    h = h_ref[...]
    z = lax.dot_general(h, wfc_ref[...], (((1,), (1,)), ((), ())),
                        preferred_element_type=jnp.float32)
    z_ref[...] = z
    st_ref[...] = lax.dot_general(z, wa_ref[...], (((1,), (1,)), ((), ())),
                                  preferred_element_type=jnp.float32)


def _zst(h, wfc, wa2):
    return pl.pallas_call(
        _zst_body,
        grid=(RB,),
        in_specs=[
            pl.BlockSpec((TCB, D), lambda i: (i, 0)),
            pl.BlockSpec((D, D), lambda i: (0, 0)),
            pl.BlockSpec((2, D), lambda i: (0, 0)),
        ],
        out_specs=[
            pl.BlockSpec((TCB, D), lambda i: (i, 0)),
            pl.BlockSpec((TCB, 2), lambda i: (i, 0)),
        ],
        out_shape=[
            jax.ShapeDtypeStruct((N, D), jnp.float32),
            jax.ShapeDtypeStruct((N, 2), jnp.float32),
        ],
    )(h, wfc, wa2)


_SC_MESH = plsc.VectorSubcoreMesh(core_axis_name="c", subcore_axis_name="s")


@functools.partial(
    pl.kernel,
    mesh=_SC_MESH,
    out_type=(
        jax.ShapeDtypeStruct((NC, N, D), jnp.float32),
        jax.ShapeDtypeStruct((N,), jnp.float32),
        jax.ShapeDtypeStruct((N,), jnp.float32),
    ),
    scratch_types=[
        pltpu.VMEM(((NCH + 1) * C,), jnp.int32),
        pltpu.VMEM((NCH + 1, C), jnp.int32),
        pltpu.VMEM((C,), jnp.float32),
        pltpu.VMEM((C,), jnp.float32),
        pltpu.VMEM((C,), jnp.float32),
        pltpu.VMEM((C,), jnp.float32),
        pltpu.VMEM((C,), jnp.float32),
        pltpu.VMEM((C,), jnp.float32),
        pltpu.VMEM((C, D), jnp.float32),
        pltpu.VMEM((C, D), jnp.float32),
        pltpu.VMEM_SHARED((N, D), jnp.float32),
        pltpu.VMEM_SHARED((N,), jnp.float32),
        pltpu.SemaphoreType.DMA,
        pltpu.SemaphoreType.DMA,
        pltpu.SemaphoreType.DMA,
        pltpu.SemaphoreType.DMA,
        pltpu.SemaphoreType.DMA,
        pltpu.SemaphoreType.DMA,
        pltpu.SemaphoreType.DMA,
        pltpu.SemaphoreType.DMA,
    ],
    compiler_params=pltpu.CompilerParams(needs_layout_passes=False),
)
def _gat_sc(src_hbm, dst2_hbm, s_hbm, t_hbm, z_hbm, zrow_hbm, zden_hbm,
            pout_hbm, pden0_hbm, pden1_hbm,
            src_f, dst2_v, sA, sB, tA, tB, wA, wB, rowsA, rowsB,
            out_sh, den_sh,
            semzA, semzB, semstA, semstB, semrA, semrB, semdA, semdB):
    cid = lax.axis_index("c")
    sid = lax.axis_index("s")
    wid = cid * NS + sid

    pltpu.sync_copy(src_hbm.at[wid], src_f)
    pltpu.sync_copy(dst2_hbm.at[wid], dst2_v)

    pltpu.sync_copy(zrow_hbm.at[pl.ds(sid * ZR, ZR)],
                    out_sh.at[pl.ds(sid * ZR, ZR)])

    @pl.when(sid == NS - 1)
    def _():
        pltpu.sync_copy(zrow_hbm.at[pl.ds(REM_OFF, REM)],
                        out_sh.at[pl.ds(REM_OFF, REM)])

    @pl.when(sid == 0)
    def _():
        pltpu.sync_copy(zden_hbm, den_sh)

    plsc.subcore_barrier()

    zeros16 = jnp.zeros((16,), jnp.int32)

    bufs = (
        (sA, tA, wA, rowsA, semzA, semstA, semrA, semdA),
        (sB, tB, wB, rowsB, semzB, semstB, semrB, semdB),
    )

    def issue_st(c, bx):
        sb, tb, semst = bx[0], bx[1], bx[5]
        pltpu.async_copy(s_hbm.at[src_f.at[pl.ds(c * C, C)]], sb, semst)
        pltpu.async_copy(t_hbm.at[dst2_v.at[c]], tb, semst)

    def issue_z(c, bx):
        rows, semz = bx[3], bx[4]
        pltpu.async_copy(z_hbm.at[src_f.at[pl.ds(c * C, C)]], rows, semz)

    def issue_gathers(c, bx):
        issue_z(c, bx)
        issue_st(c, bx)

    def wait_st(bx):
        sb, tb, _, _, _, semst, _, _ = bx
        pltpu.make_async_copy(s_hbm.at[pl.ds(0, C)], sb, semst).wait()
        pltpu.make_async_copy(t_hbm.at[pl.ds(0, C)], tb, semst).wait()

    def wait_z(bx):
        rows, semz = bx[3], bx[4]
        pltpu.make_async_copy(z_hbm.at[pl.ds(0, C)], rows, semz).wait()

    iota16 = jnp.arange(16, dtype=jnp.int32)

    def compute_w(c, bx):
        sb, tb, wb = bx[0], bx[1], bx[2]
        ebase = c * C
        for g in range(C // 16):
            sl = pl.ds(g * 16, 16)
            e = sb[sl] + tb[sl]
            e = jnp.where(e > 0.0, e, e * 0.01)
            w = jnp.exp(e)
            eid = iota16 + (ebase + g * 16)
            wb[sl] = jnp.where(eid < EPW, w, 0.0)

    UNROLL = 8

    def scale_rows(bx):
        wb, rows = bx[2], bx[3]

        def scale(j, carry):
            j0 = j * UNROLL
            for u in range(UNROLL):
                r = j0 + u
                wsplat = plsc.load_gather(wb, [zeros16 + r])
                for kk in range(D // 16):
                    sl = pl.ds(kk * 16, 16)
                    rows[r, sl] = rows[r, sl] * wsplat
            return carry

        lax.fori_loop(0, C // UNROLL, scale, 0)

    def issue_scatters(c, bx):
        wb, rows, semr, semd = bx[2], bx[3], bx[6], bx[7]
        didx = dst2_v.at[c]
        pltpu.async_copy(rows, out_sh.at[didx], semr, add=True)
        pltpu.async_copy(wb, den_sh.at[didx], semd, add=True)

    def wait_scatters(bx):
        wb, rows, semr, semd = bx[2], bx[3], bx[6], bx[7]
        pltpu.make_async_copy(rows, z_hbm.at[pl.ds(0, C)], semr).wait()
        pltpu.make_async_copy(wb, s_hbm.at[pl.ds(0, C)], semd).wait()

    def step(c, bx, by):
        issue_st(c + 1, by)
        wait_st(bx)
        compute_w(c, bx)
        wait_z(bx)
        wait_scatters(by)
        issue_z(c + 1, by)
        scale_rows(bx)
        issue_scatters(c, bx)

    issue_gathers(0, bufs[0])
    wait_st(bufs[0])
    compute_w(0, bufs[0])
    wait_z(bufs[0])
    issue_gathers(1, bufs[1])
    scale_rows(bufs[0])
    issue_scatters(0, bufs[0])

    def pair(p, carry):
        step(2 * p + 1, bufs[1], bufs[0])
        step(2 * p + 2, bufs[0], bufs[1])
        return carry

    lax.fori_loop(0, (NCH - 1) // 2, pair, 0)

    wait_scatters(bufs[0])
    wait_st(bufs[1])
    wait_z(bufs[1])

    plsc.subcore_barrier()
    pltpu.sync_copy(out_sh.at[pl.ds(sid * ZR, ZR)],
                    pout_hbm.at[cid, pl.ds(sid * ZR, ZR)])

    @pl.when(sid == NS - 1)
    def _():
        pltpu.sync_copy(out_sh.at[pl.ds(REM_OFF, REM)],
                        pout_hbm.at[cid, pl.ds(REM_OFF, REM)])

    @pl.when(jnp.logical_and(sid == 0, cid == 0))
    def _():
        pltpu.sync_copy(den_sh, pden0_hbm)

    @pl.when(jnp.logical_and(sid == 0, cid == 1))
    def _():
        pltpu.sync_copy(den_sh, pden1_hbm)


def _fin_body(pout_ref, pd0_ref, pd1_ref, out_ref):
    p = pout_ref[0] + pout_ref[1]
    d = pd0_ref[...] + pd1_ref[...]
    out_ref[...] = p / jnp.maximum(d, 1e-16)


def _finish(pout, pd0, pd1):
    return pl.pallas_call(
        _fin_body,
        grid=(RB,),
        in_specs=[
            pl.BlockSpec((NC, TCB, D), lambda i: (0, i, 0)),
            pl.BlockSpec((TCB, 1), lambda i: (i, 0)),
            pl.BlockSpec((TCB, 1), lambda i: (i, 0)),
        ],
        out_specs=pl.BlockSpec((TCB, D), lambda i: (i, 0)),
        out_shape=jax.ShapeDtypeStruct((N, D), jnp.float32),
    )(pout, pd0, pd1)


def kernel(h, edge_index, W_fc, W_attn):
    z, st = _zst(h, W_fc, W_attn.reshape(2, D))
    pad = jnp.zeros((NW, PADW), jnp.int32)
    src = jnp.concatenate([edge_index[0].reshape(NW, EPW), pad], axis=1)
    dst2 = jnp.concatenate([edge_index[1].reshape(NW, EPW), pad],
                           axis=1).reshape(NW, NCH + 1, C)
    zrow = jnp.zeros((N, D), jnp.float32)
    zden = jnp.zeros((N,), jnp.float32)
    pout, pden0, pden1 = _gat_sc(src, dst2, st[:, 0], st[:, 1], z,
                                 zrow, zden)
    return _finish(pout, pden0.reshape(N, 1), pden1.reshape(N, 1))
